# baseline (speedup 1.0000x reference)
import functools

import jax
import jax.numpy as jnp
from jax import lax
from jax.experimental import pallas as pl
from jax.experimental.pallas import tpu as pltpu
from jax.experimental.pallas import tpu_sc as plsc

BATCH = 16384
GRID = 6 * 5
TOK = BATCH * GRID
EMB = 32
FB = 3
OUT_D = EMB + FB

NC = 2
NS = 16
NW = NC * NS
TPW = TOK // NW
CHUNK = 1024
IDX_W = 128
IDX_R = CHUNK // IDX_W
NCHUNK = TPW // CHUNK


def _build():
    mesh = plsc.VectorSubcoreMesh(core_axis_name="c", subcore_axis_name="s")

    @functools.partial(
        pl.kernel,
        mesh=mesh,
        out_type=jax.ShapeDtypeStruct((TOK, OUT_D), jnp.float32),
        compiler_params=pltpu.CompilerParams(use_tc_tiling_on_sc=False),
        scratch_types=[
            pltpu.VMEM((IDX_R, IDX_W), jnp.int32),
            pltpu.VMEM((CHUNK, EMB), jnp.float32),
            pltpu.VMEM((CHUNK, FB), jnp.float32),
            pltpu.SemaphoreType.DMA,
        ],
    )
    def sc_kernel(letters_hbm, fb_hbm, table_hbm, out_hbm, idx_v, rows_v, fb_v, sem):
        wid = lax.axis_index("s") * NC + lax.axis_index("c")
        wbase = wid * TPW

        def body(i, carry):
            base = wbase + i * CHUNK
            row0 = pl.multiple_of(base // IDX_W, 8)
            pltpu.sync_copy(letters_hbm.at[pl.ds(row0, IDX_R)], idx_v)
            for j in range(IDX_R):
                pltpu.async_copy(
                    table_hbm.at[idx_v.at[j]],
                    rows_v.at[pl.ds(j * IDX_W, IDX_W)],
                    sem,
                )
            pltpu.sync_copy(fb_hbm.at[pl.ds(base, CHUNK)], fb_v)
            for j in range(IDX_R):
                pltpu.make_async_copy(
                    table_hbm.at[idx_v.at[j]],
                    rows_v.at[pl.ds(j * IDX_W, IDX_W)],
                    sem,
                ).wait()
            pltpu.sync_copy(rows_v, out_hbm.at[pl.ds(base, CHUNK), pl.ds(0, EMB)])
            pltpu.sync_copy(fb_v, out_hbm.at[pl.ds(base, CHUNK), pl.ds(EMB, FB)])
            return carry

        lax.fori_loop(0, NCHUNK, body, 0)

    return sc_kernel


_sc_kernel = _build()


@jax.jit
def kernel(letter_tensor, feedback_tensor, meta_tensor, letter_embed_table):
    letters = letter_tensor.reshape(TOK // IDX_W, IDX_W)
    fb = feedback_tensor.reshape(TOK, FB)
    out = _sc_kernel(letters, fb, letter_embed_table)
    return out.reshape(BATCH, 6, GRID // 6, OUT_D), meta_tensor


# speedup vs baseline: 1.3812x; 1.3812x over previous
import functools

import jax
import jax.numpy as jnp
from jax import lax
from jax.experimental import pallas as pl
from jax.experimental.pallas import tpu as pltpu
from jax.experimental.pallas import tpu_sc as plsc

BATCH = 16384
GRID = 6 * 5
TOK = BATCH * GRID
EMB = 32
FB = 3
OUT_D = EMB + FB

NC = 2
NS = 16
NW = NC * NS
TPW = TOK // NW
CHUNK = 1024
IDX_W = 128
IDX_R = CHUNK // IDX_W
NCHUNK = TPW // CHUNK
NGROUP = CHUNK // 16


def _build():
    mesh = plsc.VectorSubcoreMesh(core_axis_name="c", subcore_axis_name="s")

    @functools.partial(
        pl.kernel,
        mesh=mesh,
        out_type=jax.ShapeDtypeStruct((TOK * OUT_D,), jnp.float32),
        compiler_params=pltpu.CompilerParams(
            use_tc_tiling_on_sc=False, needs_layout_passes=False
        ),
        scratch_types=[
            pltpu.VMEM((IDX_R, IDX_W), jnp.int32),
            pltpu.VMEM((CHUNK, EMB), jnp.float32),
            pltpu.VMEM((CHUNK * OUT_D,), jnp.float32),
            pltpu.VMEM((CHUNK * FB,), jnp.float32),
            pltpu.SemaphoreType.DMA,
        ],
    )
    def sc_kernel(letters_hbm, fb_hbm, table_hbm, out_hbm,
                  idx_v, rows_v, out_v, fb_v, sem):
        wid = lax.axis_index("s") * NC + lax.axis_index("c")
        wbase = wid * TPW

        lane = lax.iota(jnp.int32, 16)
        fb_pat = []
        for p in range(FB):
            m = lane + (p * 16)
            q = lax.shift_right_logical(m * 21846, 16)
            fb_pat.append(q * OUT_D + (m - q * FB) + EMB)

        def chunk_body(i, carry):
            base = pl.multiple_of(wbase + i * CHUNK, CHUNK)
            row0 = pl.multiple_of(base // IDX_W, IDX_R)
            pltpu.sync_copy(letters_hbm.at[pl.ds(row0, IDX_R)], idx_v)
            for j in range(IDX_R):
                pltpu.async_copy(
                    table_hbm.at[idx_v.at[j]],
                    rows_v.at[pl.ds(j * IDX_W, IDX_W)],
                    sem,
                )
            pltpu.sync_copy(fb_hbm.at[pl.ds(base * FB, CHUNK * FB)], fb_v)
            for j in range(IDX_R):
                pltpu.make_async_copy(
                    table_hbm.at[idx_v.at[j]],
                    rows_v.at[pl.ds(j * IDX_W, IDX_W)],
                    sem,
                ).wait()

            def group_body(g, carry2):
                t0 = g * 16
                o0 = g * (16 * OUT_D)
                for t in range(16):
                    tok = t0 + t
                    dst = o0 + t * OUT_D
                    out_v[pl.ds(dst, 16)] = rows_v[tok, pl.ds(0, 16)]
                    out_v[pl.ds(dst + 16, 16)] = rows_v[tok, pl.ds(16, 16)]
                f0 = g * (16 * FB)
                for p in range(FB):
                    vals = fb_v[pl.ds(f0 + p * 16, 16)]
                    plsc.store_scatter(out_v, [fb_pat[p] + o0], vals)
                return carry2

            lax.fori_loop(0, NGROUP, group_body, 0)

            pltpu.sync_copy(out_v, out_hbm.at[pl.ds(base * OUT_D, CHUNK * OUT_D)])
            return carry

        lax.fori_loop(0, NCHUNK, chunk_body, 0)

    return sc_kernel


_sc_kernel = _build()


@jax.jit
def kernel(letter_tensor, feedback_tensor, meta_tensor, letter_embed_table):
    letters = letter_tensor.reshape(TOK // IDX_W, IDX_W)
    fb = feedback_tensor.reshape(TOK * FB)
    out = _sc_kernel(letters, fb, letter_embed_table)
    return out.reshape(BATCH, 6, GRID // 6, OUT_D), meta_tensor
